# baseline (speedup 1.0000x reference)
import functools

import jax
import jax.numpy as jnp
from jax import lax
from jax.experimental import pallas as pl
from jax.experimental.pallas import tpu as pltpu
from jax.experimental.pallas import tpu_sc as plsc

_NC = 2
_NS = 16
_L = 16
_NW = _NC * _NS

_D = 128
_DC = _D // _L
_G = 64
_MARGIN = 1.0
_WPACK = 6 * _D + 2 * _L


def _log1p_exp(u):
    up1 = 1.0 + u
    v = 2.0 * u / (2.0 + u)
    v = v - 1.0 + up1 * jnp.exp(-v)
    v = v - 1.0 + up1 * jnp.exp(-v)
    return v


def _sc_kernel_fn(B):
    BPW = B // _NW
    NCH = BPW // _G
    NGRP = _G // _L
    NIG = BPW // _L

    mesh = plsc.VectorSubcoreMesh(
        core_axis_name="c", subcore_axis_name="s",
        num_cores=_NC, num_subcores=_NS)

    def body(ptrip, ntrip, etab, rtab, wpack,
             o_part,
             trip_v, idx_h, idx_r, idx_t,
             wp_v, part_v,
             hrows0, rrows0, trows0, hrows1, rrows1, trows1,
             sem0, sem1, sem2):
        wid = lax.axis_index("s") * _NC + lax.axis_index("c")
        base = wid * BPW

        zero16 = jnp.zeros((_L,), jnp.float32)
        lanes = lax.iota(jnp.int32, _L)
        bufs = ((hrows0, rrows0, trows0), (hrows1, rrows1, trows1))
        sems = (sem0, sem1)

        cp_w = pltpu.async_copy(wpack, wp_v, sem2)
        cp_slab = pltpu.async_copy(ptrip.at[pl.ds(base, BPW)], trip_v, sem2)
        cp_w.wait()
        cp_slab.wait()

        def extract_idx():
            def ext(j, _):
                rows = j * _L + lanes
                for col, dst in ((0, idx_h), (1, idx_r), (2, idx_t)):
                    cv = jnp.full((_L,), col, jnp.int32)
                    v = plsc.load_gather(trip_v, [rows, cv])
                    dst[pl.ds(j * _L, _L)] = v
                return 0

            lax.fori_loop(0, NIG, ext, 0)

        extract_idx()

        bgv = wp_v[pl.ds(6 * _D, _L)]
        bdv = wp_v[pl.ds(6 * _D + _L, _L)]

        def fire(i, b):
            c = i % NCH
            off = c * _G
            hb, rb, tb = bufs[b]
            sem = sems[b]
            return (
                pltpu.async_copy(etab.at[idx_h.at[pl.ds(off, _G)]], hb, sem),
                pltpu.async_copy(rtab.at[idx_r.at[pl.ds(off, _G)]], rb, sem),
                pltpu.async_copy(etab.at[idx_t.at[pl.ds(off, _G)]], tb, sem),
            )

        def compute(i, b, acc):
            is_pos = i < NCH
            woff = 0 if is_pos else 3 * _D
            hb, rb, tb = bufs[b]
            wch = [(wp_v[pl.ds(woff + cc * _L, _L)],
                    wp_v[pl.ds(woff + _D + cc * _L, _L)],
                    wp_v[pl.ds(woff + 2 * _D + cc * _L, _L)])
                   for cc in range(_DC)]
            c = i % NCH

            def grp_body(g, acc):
                a1, a2 = acc

                def tri_body(j, carry):
                    od16, ow16 = carry
                    row = g * _L + j
                    accd = zero16
                    accw = zero16
                    for cc in range(_DC):
                        hv = hb[row, pl.ds(cc * _L, _L)]
                        rv = rb[row, pl.ds(cc * _L, _L)]
                        tv = tb[row, pl.ds(cc * _L, _L)]
                        accd = accd + jnp.abs(hv + rv - tv)
                        wh, wr, wt = wch[cc]
                        accw = accw + hv * wh + rv * wr + tv * wt
                    dsum = jnp.sum(accd)
                    wsum = jnp.sum(accw)
                    sel = lanes == j
                    od16 = jnp.where(sel, dsum, od16)
                    ow16 = jnp.where(sel, wsum, ow16)
                    return od16, ow16

                od16, ow16 = lax.fori_loop(
                    0, _L, tri_body, (zero16, zero16))
                if is_pos:
                    x = od16 - _MARGIN
                    u = jnp.exp(-jnp.abs(x))
                    ps = jnp.maximum(x, 0.0) + _log1p_exp(u)
                    dout = 1.0 / (1.0 + jnp.exp(-(ow16 + bdv)))
                    a1n = a1 + dout * ps
                    a2n = a2 + dout
                else:
                    mask = ((ow16 + bgv) > 0.0).astype(jnp.float32)
                    a1n = a1 + od16 * mask
                    a2n = a2 + mask
                return a1n, a2n

            return lax.fori_loop(0, NGRP, grp_body, acc)

        accp = (zero16, zero16)
        accn = (zero16, zero16)
        pending = fire(0, 0)
        slab_cp = None
        for i in range(2 * NCH):
            if i == NCH - 1:
                slab_cp.wait()
                extract_idx()
            nxt = fire(i + 1, (i + 1) % 2) if i + 1 < 2 * NCH else None
            if i == NCH - 2:
                slab_cp = pltpu.async_copy(
                    ntrip.at[pl.ds(base, BPW)], trip_v, sem2)
            for cp in pending:
                cp.wait()
            if i < NCH:
                accp = compute(i, i % 2, accp)
            else:
                accn = compute(i, i % 2, accn)
            pending = nxt

        a = jnp.sum(accp[0])
        s = jnp.sum(accp[1])
        m = jnp.sum(accn[0])
        c = jnp.sum(accn[1])
        res = jnp.where(lanes == 0, a, 0.0)
        res = jnp.where(lanes == 1, s, res)
        res = jnp.where(lanes == 2, m, res)
        res = jnp.where(lanes == 3, c, res)
        part_v[pl.ds(0, _L)] = res
        pltpu.sync_copy(part_v, o_part.at[wid])

    f32 = jnp.float32
    i32 = jnp.int32
    return pl.kernel(
        body,
        out_type=jax.ShapeDtypeStruct((_NW, _L), f32),
        mesh=mesh,
        compiler_params=pltpu.CompilerParams(needs_layout_passes=False),
        scratch_types=[
            pltpu.VMEM((BPW, 3), i32),
            pltpu.VMEM((BPW,), i32),
            pltpu.VMEM((BPW,), i32),
            pltpu.VMEM((BPW,), i32),
            pltpu.VMEM((_WPACK,), f32),
            pltpu.VMEM((_L,), f32),
            pltpu.VMEM((_G, _D), f32),
            pltpu.VMEM((_G, _D), f32),
            pltpu.VMEM((_G, _D), f32),
            pltpu.VMEM((_G, _D), f32),
            pltpu.VMEM((_G, _D), f32),
            pltpu.VMEM((_G, _D), f32),
            pltpu.SemaphoreType.DMA,
            pltpu.SemaphoreType.DMA,
            pltpu.SemaphoreType.DMA,
        ],
    )


def _combine_body(part_ref, out_ref):
    p = part_ref[...]
    a = jnp.sum(p[:, 0])
    s = jnp.sum(p[:, 1])
    m = jnp.sum(p[:, 2])
    c = jnp.sum(p[:, 3])
    out_ref[...] = (a + (m / c) * s).reshape(1, 1)


def _combine_call(part):
    return pl.pallas_call(
        _combine_body,
        out_shape=jax.ShapeDtypeStruct((1, 1), jnp.float32),
        in_specs=[pl.BlockSpec(memory_space=pltpu.VMEM)],
        out_specs=pl.BlockSpec(memory_space=pltpu.VMEM),
    )(part)


def kernel(positive_triples, negative_triples, entity_table, relation_table,
           Wg, bg, Wd, bd):
    B = positive_triples.shape[0]
    wpack = jnp.concatenate([
        Wd[:, 0], Wg[:, 0],
        jnp.broadcast_to(bg[0:1], (_L,)),
        jnp.broadcast_to(bd[0:1], (_L,)),
    ]).astype(jnp.float32)

    part = _sc_kernel_fn(B)(
        positive_triples, negative_triples,
        entity_table, relation_table, wpack)

    out = _combine_call(part)
    return out[0, 0]


# speedup vs baseline: 1.2987x; 1.2987x over previous
import functools

import jax
import jax.numpy as jnp
from jax import lax
from jax.experimental import pallas as pl
from jax.experimental.pallas import tpu as pltpu
from jax.experimental.pallas import tpu_sc as plsc

_NC = 2
_NS = 16
_L = 16
_NW = _NC * _NS

_D = 128
_DC = _D // _L
_G = 64
_MARGIN = 1.0
_WPACK = 6 * _D + 2 * _L


def _log1p_exp(u):
    up1 = 1.0 + u
    v = 2.0 * u / (2.0 + u)
    v = v - 1.0 + up1 * jnp.exp(-v)
    v = v - 1.0 + up1 * jnp.exp(-v)
    return v


def _sc_kernel_fn(B):
    BPW = B // _NW
    NCH = BPW // _G
    NGRP = _G // _L
    NIG = BPW // _L

    mesh = plsc.VectorSubcoreMesh(
        core_axis_name="c", subcore_axis_name="s",
        num_cores=_NC, num_subcores=_NS)

    def body(idxpack, etab, rtab, wpack,
             o_part,
             idx_ph, idx_pr, idx_pt, idx_nh, idx_nr, idx_nt,
             wp_v, part_v,
             hrows0, rrows0, trows0, hrows1, rrows1, trows1,
             sem0, sem1, sem2, sem3):
        wid = lax.axis_index("s") * _NC + lax.axis_index("c")
        base = wid * BPW

        zero16 = jnp.zeros((_L,), jnp.float32)
        lanes = lax.iota(jnp.int32, _L)
        bufs = ((hrows0, rrows0, trows0), (hrows1, rrows1, trows1))
        sems = (sem0, sem1)

        pos_cps = (
            pltpu.async_copy(idxpack.at[pl.ds(0 * B + base, BPW)],
                             idx_ph, sem3),
            pltpu.async_copy(idxpack.at[pl.ds(1 * B + base, BPW)],
                             idx_pr, sem3),
            pltpu.async_copy(idxpack.at[pl.ds(2 * B + base, BPW)],
                             idx_pt, sem3),
        )
        rest_cps = (
            pltpu.async_copy(idxpack.at[pl.ds(3 * B + base, BPW)],
                             idx_nh, sem2),
            pltpu.async_copy(idxpack.at[pl.ds(4 * B + base, BPW)],
                             idx_nr, sem2),
            pltpu.async_copy(idxpack.at[pl.ds(5 * B + base, BPW)],
                             idx_nt, sem2),
            pltpu.async_copy(wpack, wp_v, sem2),
        )

        bgv_off = 6 * _D
        bdv_off = 6 * _D + _L

        idxs = ((idx_ph, idx_pr, idx_pt), (idx_nh, idx_nr, idx_nt))

        def fire(i, b):
            ih, ir, it = idxs[0] if i < NCH else idxs[1]
            off = (i % NCH) * _G
            hb, rb, tb = bufs[b]
            sem = sems[b]
            return (
                pltpu.async_copy(etab.at[ih.at[pl.ds(off, _G)]], hb, sem),
                pltpu.async_copy(rtab.at[ir.at[pl.ds(off, _G)]], rb, sem),
                pltpu.async_copy(etab.at[it.at[pl.ds(off, _G)]], tb, sem),
            )

        def compute(i, b, acc):
            is_pos = i < NCH
            woff = 0 if is_pos else 3 * _D
            hb, rb, tb = bufs[b]
            wch = [(wp_v[pl.ds(woff + cc * _L, _L)],
                    wp_v[pl.ds(woff + _D + cc * _L, _L)],
                    wp_v[pl.ds(woff + 2 * _D + cc * _L, _L)])
                   for cc in range(_DC)]
            c = i % NCH

            def grp_body(g, acc):
                a1, a2 = acc

                def tri_body(j, carry):
                    od16, ow16 = carry
                    row = g * _L + j
                    accd = zero16
                    accw = zero16
                    for cc in range(_DC):
                        hv = hb[row, pl.ds(cc * _L, _L)]
                        rv = rb[row, pl.ds(cc * _L, _L)]
                        tv = tb[row, pl.ds(cc * _L, _L)]
                        accd = accd + jnp.abs(hv + rv - tv)
                        wh, wr, wt = wch[cc]
                        accw = accw + hv * wh + rv * wr + tv * wt
                    dsum = jnp.sum(accd)
                    wsum = jnp.sum(accw)
                    sel = lanes == j
                    od16 = jnp.where(sel, dsum, od16)
                    ow16 = jnp.where(sel, wsum, ow16)
                    return od16, ow16

                od16, ow16 = lax.fori_loop(
                    0, _L, tri_body, (zero16, zero16))
                if is_pos:
                    bdv = wp_v[pl.ds(bdv_off, _L)]
                    x = od16 - _MARGIN
                    u = jnp.exp(-jnp.abs(x))
                    ps = jnp.maximum(x, 0.0) + _log1p_exp(u)
                    dout = 1.0 / (1.0 + jnp.exp(-(ow16 + bdv)))
                    a1n = a1 + dout * ps
                    a2n = a2 + dout
                else:
                    bgv = wp_v[pl.ds(bgv_off, _L)]
                    mask = ((ow16 + bgv) > 0.0).astype(jnp.float32)
                    a1n = a1 + od16 * mask
                    a2n = a2 + mask
                return a1n, a2n

            return lax.fori_loop(0, NGRP, grp_body, acc)

        accp = (zero16, zero16)
        accn = (zero16, zero16)
        for cp in pos_cps:
            cp.wait()
        pending = fire(0, 0)
        for cp in rest_cps:
            cp.wait()
        for i in range(2 * NCH):
            nxt = fire(i + 1, (i + 1) % 2) if i + 1 < 2 * NCH else None
            for cp in pending:
                cp.wait()
            if i < NCH:
                accp = compute(i, i % 2, accp)
            else:
                accn = compute(i, i % 2, accn)
            pending = nxt

        a = jnp.sum(accp[0])
        s = jnp.sum(accp[1])
        m = jnp.sum(accn[0])
        c = jnp.sum(accn[1])
        res = jnp.where(lanes == 0, a, 0.0)
        res = jnp.where(lanes == 1, s, res)
        res = jnp.where(lanes == 2, m, res)
        res = jnp.where(lanes == 3, c, res)
        part_v[pl.ds(0, _L)] = res
        pltpu.sync_copy(part_v, o_part.at[wid])

    f32 = jnp.float32
    i32 = jnp.int32
    return pl.kernel(
        body,
        out_type=jax.ShapeDtypeStruct((_NW, _L), f32),
        mesh=mesh,
        compiler_params=pltpu.CompilerParams(needs_layout_passes=False),
        scratch_types=[
            pltpu.VMEM((BPW,), i32),
            pltpu.VMEM((BPW,), i32),
            pltpu.VMEM((BPW,), i32),
            pltpu.VMEM((BPW,), i32),
            pltpu.VMEM((BPW,), i32),
            pltpu.VMEM((BPW,), i32),
            pltpu.VMEM((_WPACK,), f32),
            pltpu.VMEM((_L,), f32),
            pltpu.VMEM((_G, _D), f32),
            pltpu.VMEM((_G, _D), f32),
            pltpu.VMEM((_G, _D), f32),
            pltpu.VMEM((_G, _D), f32),
            pltpu.VMEM((_G, _D), f32),
            pltpu.VMEM((_G, _D), f32),
            pltpu.SemaphoreType.DMA,
            pltpu.SemaphoreType.DMA,
            pltpu.SemaphoreType.DMA,
            pltpu.SemaphoreType.DMA,
        ],
    )


def _combine_body(part_ref, out_ref):
    p = part_ref[...]
    a = jnp.sum(p[:, 0])
    s = jnp.sum(p[:, 1])
    m = jnp.sum(p[:, 2])
    c = jnp.sum(p[:, 3])
    out_ref[...] = (a + (m / c) * s).reshape(1, 1)


def _combine_call(part):
    return pl.pallas_call(
        _combine_body,
        out_shape=jax.ShapeDtypeStruct((1, 1), jnp.float32),
        in_specs=[pl.BlockSpec(memory_space=pltpu.VMEM)],
        out_specs=pl.BlockSpec(memory_space=pltpu.VMEM),
    )(part)


def kernel(positive_triples, negative_triples, entity_table, relation_table,
           Wg, bg, Wd, bd):
    B = positive_triples.shape[0]
    wpack = jnp.concatenate([
        Wd[:, 0], Wg[:, 0],
        jnp.broadcast_to(bg[0:1], (_L,)),
        jnp.broadcast_to(bd[0:1], (_L,)),
    ]).astype(jnp.float32)
    idxpack = jnp.concatenate([
        positive_triples[:, 0], positive_triples[:, 1],
        positive_triples[:, 2], negative_triples[:, 0],
        negative_triples[:, 1], negative_triples[:, 2],
    ])

    part = _sc_kernel_fn(B)(
        idxpack, entity_table, relation_table, wpack)

    out = _combine_call(part)
    return out[0, 0]
